# Initial kernel scaffold; baseline (speedup 1.0000x reference)
import jax
import jax.numpy as jnp
from jax.experimental import pallas as pl


def kernel(vertices, edges, global_features, local_features, params):
    raise NotImplementedError("write your pallas kernel here")


# speedup vs baseline: 31.2652x; 31.2652x over previous
import functools

import jax
import jax.numpy as jnp
from jax import lax
from jax.experimental import pallas as pl
from jax.experimental.pallas import tpu as pltpu
from jax.experimental.pallas import tpu_sc as plsc


def _split_fc(gf, W, b):
    Bn, K = gf.shape
    N = W.shape[1]
    CB = 4096

    def body(g_ref, w_ref, b_ref, o_ref):
        o_ref[...] = (
            jnp.dot(g_ref[...], w_ref[...], preferred_element_type=jnp.float32)
            + b_ref[...]
        )

    return pl.pallas_call(
        body,
        grid=(N // CB,),
        in_specs=[
            pl.BlockSpec((Bn, K), lambda i: (0, 0)),
            pl.BlockSpec((K, CB), lambda i: (0, i)),
            pl.BlockSpec((1, CB), lambda i: (0, i)),
        ],
        out_specs=pl.BlockSpec((Bn, CB), lambda i: (0, i)),
        out_shape=jax.ShapeDtypeStruct((Bn, N), jnp.float32),
    )(gf, W, b.reshape(1, N))


def _conv1_mm(xvg, loc, Wr_vg, Wn_vg, Wr_l, Wn_l, b):
    BV, Kvg = xvg.shape
    Kl = loc.shape[1]
    F = Wr_l.shape[1]
    Vt = 512

    def body(x_ref, l_ref, wrvg, wnvg, wrl, wnl, b_ref, xr_ref, xn_ref):
        x = x_ref[...]
        l = l_ref[...]
        d = functools.partial(jnp.dot, preferred_element_type=jnp.float32)
        xr_ref[...] = d(x, wrvg[...]) + d(l, wrl[...]) + b_ref[...]
        xn_ref[...] = d(x, wnvg[...]) + d(l, wnl[...])

    return pl.pallas_call(
        body,
        grid=(BV // Vt,),
        in_specs=[
            pl.BlockSpec((Vt, Kvg), lambda i: (i, 0)),
            pl.BlockSpec((Vt, Kl), lambda i: (i, 0)),
            pl.BlockSpec((Kvg, F), lambda i: (0, 0)),
            pl.BlockSpec((Kvg, F), lambda i: (0, 0)),
            pl.BlockSpec((Kl, F), lambda i: (0, 0)),
            pl.BlockSpec((Kl, F), lambda i: (0, 0)),
            pl.BlockSpec((1, F), lambda i: (0, 0)),
        ],
        out_specs=[
            pl.BlockSpec((Vt, F), lambda i: (i, 0)),
            pl.BlockSpec((Vt, F), lambda i: (i, 0)),
        ],
        out_shape=[jax.ShapeDtypeStruct((BV, F), jnp.float32)] * 2,
    )(xvg, loc, Wr_vg, Wn_vg, Wr_l, Wn_l, b.reshape(1, F))


def _dense_agg(P, xnv):
    Vn = P.shape[0]
    BF = xnv.shape[1]
    RT = 256

    def body(p_ref, x_ref, o_ref):
        o_ref[...] = jnp.dot(
            p_ref[...], x_ref[...], preferred_element_type=jnp.float32
        )

    return pl.pallas_call(
        body,
        grid=(Vn // RT,),
        in_specs=[
            pl.BlockSpec((RT, Vn), lambda i: (i, 0)),
            pl.BlockSpec((Vn, BF), lambda i: (0, 0)),
        ],
        out_specs=pl.BlockSpec((RT, BF), lambda i: (i, 0)),
        out_shape=jax.ShapeDtypeStruct((Vn, BF), jnp.float32),
    )(P, xnv)


def _rowsum(P):
    Vn = P.shape[0]
    RT = 256

    def body(p_ref, o_ref):
        o_ref[...] = jnp.sum(p_ref[...], axis=1, keepdims=True)

    return pl.pallas_call(
        body,
        grid=(Vn // RT,),
        in_specs=[pl.BlockSpec((RT, Vn), lambda i: (i, 0))],
        out_specs=pl.BlockSpec((RT, 1), lambda i: (i, 0)),
        out_shape=jax.ShapeDtypeStruct((Vn, 1), jnp.float32),
    )(P)


def _combine(xr, agg, degr, xprev, mats, relu1, emit_t):
    BV, F = xr.shape
    Vt = 512

    def body(*refs):
        it = iter(refs)
        xr_ref = next(it)
        agg_ref = next(it)
        deg_ref = next(it)
        xp_ref = next(it) if xprev is not None else None
        wb_refs = []
        for _, bias in mats:
            wr = next(it)
            br = next(it) if bias is not None else None
            wb_refs.append((wr, br))
        outs = list(it)
        r = 1.0 / jnp.maximum(deg_ref[...], 1.0)
        t = xr_ref[...] + agg_ref[...] * r
        if relu1:
            t = jnp.maximum(t, 0.0)
        if xp_ref is not None:
            t = jnp.maximum(xp_ref[...] + t, 0.0)
        k = 0
        if emit_t:
            outs[0][...] = t
            k = 1
        for (wr, br), o_ref in zip(wb_refs, outs[k:]):
            y = jnp.dot(t, wr[...], preferred_element_type=jnp.float32)
            if br is not None:
                y = y + br[...]
            o_ref[...] = y

    in_specs = [
        pl.BlockSpec((Vt, F), lambda i: (i, 0)),
        pl.BlockSpec((Vt, F), lambda i: (i, 0)),
        pl.BlockSpec((Vt, 1), lambda i: (i, 0)),
    ]
    args = [xr, agg, degr]
    if xprev is not None:
        in_specs.append(pl.BlockSpec((Vt, xprev.shape[1]), lambda i: (i, 0)))
        args.append(xprev)
    out_shapes = []
    out_specs = []
    if emit_t:
        out_shapes.append(jax.ShapeDtypeStruct((BV, F), jnp.float32))
        out_specs.append(pl.BlockSpec((Vt, F), lambda i: (i, 0)))
    for W, bias in mats:
        Ki, Fo = W.shape
        in_specs.append(pl.BlockSpec((Ki, Fo), lambda i: (0, 0)))
        args.append(W)
        if bias is not None:
            in_specs.append(pl.BlockSpec((1, Fo), lambda i: (0, 0)))
            args.append(bias.reshape(1, Fo))
        out_shapes.append(jax.ShapeDtypeStruct((BV, Fo), jnp.float32))
        out_specs.append(pl.BlockSpec((Vt, Fo), lambda i: (i, 0)))

    res = pl.pallas_call(
        body,
        grid=(BV // Vt,),
        in_specs=in_specs,
        out_specs=out_specs,
        out_shape=out_shapes,
    )(*args)
    return res if isinstance(res, (list, tuple)) else [res]


_ECHUNK = 2048
_PASS_ROWS = 32


def _sc_build_p(dst, src, Vn):
    info = plsc.get_sparse_core_info()
    NC, NS = info.num_cores, info.num_subcores
    NW = NC * NS
    CE = _ECHUNK
    Ep = dst.shape[0]
    nch = Ep // CE
    RPW = Vn // NW
    npass = RPW // _PASS_ROWS
    mesh = plsc.VectorSubcoreMesh(core_axis_name="c", subcore_axis_name="s")

    @functools.partial(
        pl.kernel,
        out_type=jax.ShapeDtypeStruct((Vn * Vn,), jnp.float32),
        mesh=mesh,
        compiler_params=pltpu.CompilerParams(needs_layout_passes=False),
        scratch_types=[
            pltpu.VMEM((CE,), jnp.int32),
            pltpu.VMEM((CE,), jnp.int32),
            pltpu.VMEM((_PASS_ROWS * Vn,), jnp.float32),
        ],
    )
    def k(dst_hbm, src_hbm, out_hbm, dv, sv, acc):
        c = lax.axis_index("c")
        s = lax.axis_index("s")
        w = s * NC + c
        ones = jnp.ones((16,), jnp.float32)
        zero = jnp.zeros((16,), jnp.float32)

        for p in range(npass):
            rowbase = (w * npass + p) * _PASS_ROWS

            def zf(i, _):
                acc[pl.ds(i * 16, 16)] = zero
                return 0

            lax.fori_loop(0, _PASS_ROWS * Vn // 16, zf, 0)

            def chunk(kk, _):
                base = kk * CE
                pltpu.sync_copy(dst_hbm.at[pl.ds(base, CE)], dv)
                pltpu.sync_copy(src_hbm.at[pl.ds(base, CE)], sv)
                for g in range(CE // 16):
                    dstv = dv[pl.ds(g * 16, 16)]
                    srcv = sv[pl.ds(g * 16, 16)]
                    loc = dstv - rowbase
                    m = (loc >= 0) & (loc < _PASS_ROWS)
                    idx = jnp.clip(loc * Vn + srcv, 0, _PASS_ROWS * Vn - 1)
                    plsc.addupdate_scatter(acc, [idx], ones, mask=m)
                return 0

            lax.fori_loop(0, nch, chunk, 0)
            pltpu.sync_copy(
                acc, out_hbm.at[pl.ds(rowbase * Vn, _PASS_ROWS * Vn)]
            )

    return k(dst, src)


def kernel(vertices, edges, global_features, local_features, params):
    B, Vn, _ = vertices.shape
    E = edges.shape[1]
    BV = B * Vn
    p = params

    src = edges[0].astype(jnp.int32)
    dst = edges[1].astype(jnp.int32)
    pad = (-E) % _ECHUNK
    if pad:
        dst = jnp.pad(dst, (0, pad), constant_values=Vn)
        src = jnp.pad(src, (0, pad))
    P = _sc_build_p(dst, src, Vn).reshape(Vn, Vn)
    deg = _rowsum(P)
    degr = jnp.repeat(deg, B, axis=0)

    g = _split_fc(global_features, p["W_split"], p["b_split"])
    g3 = g.reshape(B, Vn, 128).transpose(1, 0, 2).reshape(BV, 128)
    v3 = vertices.transpose(1, 0, 2).reshape(BV, 3)
    xvg = jnp.pad(jnp.concatenate([v3, g3], axis=1), ((0, 0), (0, 125)))
    loc3 = local_features.transpose(1, 0, 2).reshape(BV, local_features.shape[2])
    c1 = p["conv1"]
    Wr_vg = jnp.pad(c1["Wr"][:131], ((0, 125), (0, 0)))
    Wn_vg = jnp.pad(c1["Wn"][:131], ((0, 125), (0, 0)))
    Wr_l = c1["Wr"][131:]
    Wn_l = c1["Wn"][131:]
    xr, xn = _conv1_mm(xvg, loc3, Wr_vg, Wn_vg, Wr_l, Wn_l, c1["b"])

    def agg(xm, F):
        return _dense_agg(P, xm.reshape(Vn, B * F)).reshape(BV, F)

    r0, r1 = p["res"]
    x2, xr_a, xn_a = _combine(
        xr, agg(xn, 256), degr, None,
        [(r0["Wr1"], r0["b1"]), (r0["Wn1"], None)], relu1=True, emit_t=True)
    hr, hn = _combine(
        xr_a, agg(xn_a, 256), degr, None,
        [(r0["Wr2"], r0["b2"]), (r0["Wn2"], None)], relu1=True, emit_t=False)
    x3, xr_b, xn_b = _combine(
        hr, agg(hn, 256), degr, x2,
        [(r1["Wr1"], r1["b1"]), (r1["Wn1"], None)], relu1=False, emit_t=True)
    hr2, hn2 = _combine(
        xr_b, agg(xn_b, 256), degr, None,
        [(r1["Wr2"], r1["b2"]), (r1["Wn2"], None)], relu1=True, emit_t=False)
    co = p["conv_out"]
    xr_o, xn_o = _combine(
        hr2, agg(hn2, 256), degr, x3,
        [(co["Wr"], co["b"]), (co["Wn"], None)], relu1=False, emit_t=False)
    cf = p["conv_final"]
    Wr_f = jnp.pad(cf["Wr"], ((0, 0), (0, 13)))
    Wn_f = jnp.pad(cf["Wn"], ((0, 0), (0, 13)))
    b_f = jnp.pad(cf["b"], (0, 13))
    xr_f, xn_f = _combine(
        xr_o, agg(xn_o, 64), degr, None,
        [(Wr_f, b_f), (Wn_f, None)], relu1=True, emit_t=False)
    (out16,) = _combine(
        xr_f, agg(xn_f, 16), degr, None, [], relu1=False, emit_t=True)
    return out16[:, :3].reshape(Vn, B, 3).transpose(1, 0, 2)


# speedup vs baseline: 31.4450x; 1.0057x over previous
import functools

import jax
import jax.numpy as jnp
from jax import lax
from jax.experimental import pallas as pl
from jax.experimental.pallas import tpu as pltpu
from jax.experimental.pallas import tpu_sc as plsc


def _split_fc(gf, W, b):
    Bn, K = gf.shape
    N = W.shape[1]
    CB = 4096

    def body(g_ref, w_ref, b_ref, o_ref):
        o_ref[...] = (
            jnp.dot(g_ref[...], w_ref[...], preferred_element_type=jnp.float32)
            + b_ref[...]
        )

    return pl.pallas_call(
        body,
        grid=(N // CB,),
        in_specs=[
            pl.BlockSpec((Bn, K), lambda i: (0, 0)),
            pl.BlockSpec((K, CB), lambda i: (0, i)),
            pl.BlockSpec((1, CB), lambda i: (0, i)),
        ],
        out_specs=pl.BlockSpec((Bn, CB), lambda i: (0, i)),
        out_shape=jax.ShapeDtypeStruct((Bn, N), jnp.float32),
    )(gf, W, b.reshape(1, N))


def _conv1_mm(xvg, loc, Wr_vg, Wn_vg, Wr_l, Wn_l, b):
    BV, Kvg = xvg.shape
    Kl = loc.shape[1]
    F = Wr_l.shape[1]
    Vt = 512

    def body(x_ref, l_ref, wrvg, wnvg, wrl, wnl, b_ref, xr_ref, xn_ref):
        x = x_ref[...]
        l = l_ref[...]
        d = functools.partial(jnp.dot, preferred_element_type=jnp.float32)
        xr_ref[...] = d(x, wrvg[...]) + d(l, wrl[...]) + b_ref[...]
        xn_ref[...] = d(x, wnvg[...]) + d(l, wnl[...])

    return pl.pallas_call(
        body,
        grid=(BV // Vt,),
        in_specs=[
            pl.BlockSpec((Vt, Kvg), lambda i: (i, 0)),
            pl.BlockSpec((Vt, Kl), lambda i: (i, 0)),
            pl.BlockSpec((Kvg, F), lambda i: (0, 0)),
            pl.BlockSpec((Kvg, F), lambda i: (0, 0)),
            pl.BlockSpec((Kl, F), lambda i: (0, 0)),
            pl.BlockSpec((Kl, F), lambda i: (0, 0)),
            pl.BlockSpec((1, F), lambda i: (0, 0)),
        ],
        out_specs=[
            pl.BlockSpec((Vt, F), lambda i: (i, 0)),
            pl.BlockSpec((Vt, F), lambda i: (i, 0)),
        ],
        out_shape=[jax.ShapeDtypeStruct((BV, F), jnp.float32)] * 2,
    )(xvg, loc, Wr_vg, Wn_vg, Wr_l, Wn_l, b.reshape(1, F))


def _dense_agg(P, xnv):
    Vn = P.shape[0]
    BF = xnv.shape[1]
    RT = 256

    def body(p_ref, x_ref, o_ref):
        o_ref[...] = jnp.dot(
            p_ref[...], x_ref[...], preferred_element_type=jnp.float32
        )

    return pl.pallas_call(
        body,
        grid=(Vn // RT,),
        in_specs=[
            pl.BlockSpec((RT, Vn), lambda i: (i, 0)),
            pl.BlockSpec((Vn, BF), lambda i: (0, 0)),
        ],
        out_specs=pl.BlockSpec((RT, BF), lambda i: (i, 0)),
        out_shape=jax.ShapeDtypeStruct((Vn, BF), jnp.float32),
    )(P, xnv)


def _rowsum(P):
    Vn = P.shape[0]
    RT = 256

    def body(p_ref, o_ref):
        o_ref[...] = jnp.sum(p_ref[...], axis=1, keepdims=True)

    return pl.pallas_call(
        body,
        grid=(Vn // RT,),
        in_specs=[pl.BlockSpec((RT, Vn), lambda i: (i, 0))],
        out_specs=pl.BlockSpec((RT, 1), lambda i: (i, 0)),
        out_shape=jax.ShapeDtypeStruct((Vn, 1), jnp.float32),
    )(P)


def _combine(xr, agg, degr, xprev, mats, relu1, emit_t):
    BV, F = xr.shape
    Vt = 512

    def body(*refs):
        it = iter(refs)
        xr_ref = next(it)
        agg_ref = next(it)
        deg_ref = next(it)
        xp_ref = next(it) if xprev is not None else None
        wb_refs = []
        for _, bias in mats:
            wr = next(it)
            br = next(it) if bias is not None else None
            wb_refs.append((wr, br))
        outs = list(it)
        r = 1.0 / jnp.maximum(deg_ref[...], 1.0)
        t = xr_ref[...] + agg_ref[...] * r
        if relu1:
            t = jnp.maximum(t, 0.0)
        if xp_ref is not None:
            t = jnp.maximum(xp_ref[...] + t, 0.0)
        k = 0
        if emit_t:
            outs[0][...] = t
            k = 1
        for (wr, br), o_ref in zip(wb_refs, outs[k:]):
            y = jnp.dot(t, wr[...], preferred_element_type=jnp.float32)
            if br is not None:
                y = y + br[...]
            o_ref[...] = y

    in_specs = [
        pl.BlockSpec((Vt, F), lambda i: (i, 0)),
        pl.BlockSpec((Vt, F), lambda i: (i, 0)),
        pl.BlockSpec((Vt, 1), lambda i: (i, 0)),
    ]
    args = [xr, agg, degr]
    if xprev is not None:
        in_specs.append(pl.BlockSpec((Vt, xprev.shape[1]), lambda i: (i, 0)))
        args.append(xprev)
    out_shapes = []
    out_specs = []
    if emit_t:
        out_shapes.append(jax.ShapeDtypeStruct((BV, F), jnp.float32))
        out_specs.append(pl.BlockSpec((Vt, F), lambda i: (i, 0)))
    for W, bias in mats:
        Ki, Fo = W.shape
        in_specs.append(pl.BlockSpec((Ki, Fo), lambda i: (0, 0)))
        args.append(W)
        if bias is not None:
            in_specs.append(pl.BlockSpec((1, Fo), lambda i: (0, 0)))
            args.append(bias.reshape(1, Fo))
        out_shapes.append(jax.ShapeDtypeStruct((BV, Fo), jnp.float32))
        out_specs.append(pl.BlockSpec((Vt, Fo), lambda i: (i, 0)))

    res = pl.pallas_call(
        body,
        grid=(BV // Vt,),
        in_specs=in_specs,
        out_specs=out_specs,
        out_shape=out_shapes,
    )(*args)
    return res if isinstance(res, (list, tuple)) else [res]


_EPAD = 128
_PASS_ROWS = 32


def _sc_build_p(dst, src, Vn):
    info = plsc.get_sparse_core_info()
    NC, NS = info.num_cores, info.num_subcores
    NW = NC * NS
    Ep = dst.shape[0]
    RPW = Vn // NW
    npass = RPW // _PASS_ROWS
    G = 8
    ng = Ep // (16 * G)
    mesh = plsc.VectorSubcoreMesh(core_axis_name="c", subcore_axis_name="s")

    @functools.partial(
        pl.kernel,
        out_type=jax.ShapeDtypeStruct((Vn * Vn,), jnp.float32),
        mesh=mesh,
        compiler_params=pltpu.CompilerParams(needs_layout_passes=False),
        scratch_types=[
            pltpu.VMEM((Ep,), jnp.int32),
            pltpu.VMEM((Ep,), jnp.int32),
            pltpu.VMEM((_PASS_ROWS * Vn,), jnp.float32),
            pltpu.SemaphoreType.DMA,
            pltpu.SemaphoreType.DMA,
        ],
    )
    def k(dst_hbm, src_hbm, out_hbm, dv, sv, acc, sem1, sem2):
        c = lax.axis_index("c")
        s = lax.axis_index("s")
        w = s * NC + c
        ones = jnp.ones((16,), jnp.float32)
        zero = jnp.zeros((16,), jnp.float32)

        cp1 = pltpu.async_copy(dst_hbm, dv, sem1)
        cp2 = pltpu.async_copy(src_hbm, sv, sem2)
        cp1.wait()
        cp2.wait()

        for p in range(npass):
            rowbase = (w * npass + p) * _PASS_ROWS

            def zf(i, _):
                for u in range(8):
                    acc[pl.ds(i * 128 + u * 16, 16)] = zero
                return 0

            lax.fori_loop(0, _PASS_ROWS * Vn // 128, zf, 0)

            def grp(i, _):
                for u in range(G):
                    off = i * (16 * G) + u * 16
                    dstv = dv[pl.ds(off, 16)]
                    srcv = sv[pl.ds(off, 16)]
                    loc = dstv - rowbase
                    m = (loc >= 0) & (loc < _PASS_ROWS)
                    idx = jnp.clip(loc * Vn + srcv, 0, _PASS_ROWS * Vn - 1)
                    plsc.addupdate_scatter(acc, [idx], ones, mask=m)
                return 0

            lax.fori_loop(0, ng, grp, 0)
            pltpu.sync_copy(
                acc, out_hbm.at[pl.ds(rowbase * Vn, _PASS_ROWS * Vn)]
            )

    return k(dst, src)


def kernel(vertices, edges, global_features, local_features, params):
    B, Vn, _ = vertices.shape
    E = edges.shape[1]
    BV = B * Vn
    p = params

    src = edges[0].astype(jnp.int32)
    dst = edges[1].astype(jnp.int32)
    pad = (-E) % _EPAD
    if pad:
        dst = jnp.pad(dst, (0, pad), constant_values=Vn)
        src = jnp.pad(src, (0, pad))
    P = _sc_build_p(dst, src, Vn).reshape(Vn, Vn)
    deg = _rowsum(P)
    degr = jnp.repeat(deg, B, axis=0)

    g = _split_fc(global_features, p["W_split"], p["b_split"])
    g3 = g.reshape(B, Vn, 128).transpose(1, 0, 2).reshape(BV, 128)
    v3 = vertices.transpose(1, 0, 2).reshape(BV, 3)
    xvg = jnp.pad(jnp.concatenate([v3, g3], axis=1), ((0, 0), (0, 125)))
    loc3 = local_features.transpose(1, 0, 2).reshape(BV, local_features.shape[2])
    c1 = p["conv1"]
    Wr_vg = jnp.pad(c1["Wr"][:131], ((0, 125), (0, 0)))
    Wn_vg = jnp.pad(c1["Wn"][:131], ((0, 125), (0, 0)))
    Wr_l = c1["Wr"][131:]
    Wn_l = c1["Wn"][131:]
    xr, xn = _conv1_mm(xvg, loc3, Wr_vg, Wn_vg, Wr_l, Wn_l, c1["b"])

    def agg(xm, F):
        return _dense_agg(P, xm.reshape(Vn, B * F)).reshape(BV, F)

    r0, r1 = p["res"]
    x2, xr_a, xn_a = _combine(
        xr, agg(xn, 256), degr, None,
        [(r0["Wr1"], r0["b1"]), (r0["Wn1"], None)], relu1=True, emit_t=True)
    hr, hn = _combine(
        xr_a, agg(xn_a, 256), degr, None,
        [(r0["Wr2"], r0["b2"]), (r0["Wn2"], None)], relu1=True, emit_t=False)
    x3, xr_b, xn_b = _combine(
        hr, agg(hn, 256), degr, x2,
        [(r1["Wr1"], r1["b1"]), (r1["Wn1"], None)], relu1=False, emit_t=True)
    hr2, hn2 = _combine(
        xr_b, agg(xn_b, 256), degr, None,
        [(r1["Wr2"], r1["b2"]), (r1["Wn2"], None)], relu1=True, emit_t=False)
    co = p["conv_out"]
    xr_o, xn_o = _combine(
        hr2, agg(hn2, 256), degr, x3,
        [(co["Wr"], co["b"]), (co["Wn"], None)], relu1=False, emit_t=False)
    cf = p["conv_final"]
    Wr_f = jnp.pad(cf["Wr"], ((0, 0), (0, 13)))
    Wn_f = jnp.pad(cf["Wn"], ((0, 0), (0, 13)))
    b_f = jnp.pad(cf["b"], (0, 13))
    xr_f, xn_f = _combine(
        xr_o, agg(xn_o, 64), degr, None,
        [(Wr_f, b_f), (Wn_f, None)], relu1=True, emit_t=False)
    (out16,) = _combine(
        xr_f, agg(xn_f, 16), degr, None, [], relu1=False, emit_t=True)
    return out16[:, :3].reshape(Vn, B, 3).transpose(1, 0, 2)


# speedup vs baseline: 46.1996x; 1.4692x over previous
import functools

import jax
import jax.numpy as jnp
from jax import lax
from jax.experimental import pallas as pl
from jax.experimental.pallas import tpu as pltpu
from jax.experimental.pallas import tpu_sc as plsc


def _split_fc(gf, W, b):
    Bn, K = gf.shape
    N = W.shape[1]
    CB = 4096

    def body(g_ref, w_ref, b_ref, o_ref):
        o_ref[...] = (
            jnp.dot(g_ref[...], w_ref[...], preferred_element_type=jnp.float32)
            + b_ref[...]
        )

    return pl.pallas_call(
        body,
        grid=(N // CB,),
        in_specs=[
            pl.BlockSpec((Bn, K), lambda i: (0, 0)),
            pl.BlockSpec((K, CB), lambda i: (0, i)),
            pl.BlockSpec((1, CB), lambda i: (0, i)),
        ],
        out_specs=pl.BlockSpec((Bn, CB), lambda i: (0, i)),
        out_shape=jax.ShapeDtypeStruct((Bn, N), jnp.float32),
    )(gf, W, b.reshape(1, N))


def _conv1_mm(xvg, loc, Wr_vg, Wn_vg, Wr_l, Wn_l, b):
    BV, Kvg = xvg.shape
    Kl = loc.shape[1]
    F = Wr_l.shape[1]
    Vt = 512

    def body(x_ref, l_ref, wrvg, wnvg, wrl, wnl, b_ref, xr_ref, xn_ref):
        x = x_ref[...]
        l = l_ref[...]
        d = functools.partial(jnp.dot, preferred_element_type=jnp.float32)
        xr_ref[...] = d(x, wrvg[...]) + d(l, wrl[...]) + b_ref[...]
        xn_ref[...] = (d(x, wnvg[...]) + d(l, wnl[...])).astype(jnp.bfloat16)

    return pl.pallas_call(
        body,
        grid=(BV // Vt,),
        in_specs=[
            pl.BlockSpec((Vt, Kvg), lambda i: (i, 0)),
            pl.BlockSpec((Vt, Kl), lambda i: (i, 0)),
            pl.BlockSpec((Kvg, F), lambda i: (0, 0)),
            pl.BlockSpec((Kvg, F), lambda i: (0, 0)),
            pl.BlockSpec((Kl, F), lambda i: (0, 0)),
            pl.BlockSpec((Kl, F), lambda i: (0, 0)),
            pl.BlockSpec((1, F), lambda i: (0, 0)),
        ],
        out_specs=[
            pl.BlockSpec((Vt, F), lambda i: (i, 0)),
            pl.BlockSpec((Vt, F), lambda i: (i, 0)),
        ],
        out_shape=[
            jax.ShapeDtypeStruct((BV, F), jnp.float32),
            jax.ShapeDtypeStruct((BV, F), jnp.bfloat16),
        ],
    )(xvg, loc, Wr_vg, Wn_vg, Wr_l, Wn_l, b.reshape(1, F))


def _rowsum(P):
    Vn = P.shape[0]
    RT = 256

    def body(p_ref, o_ref):
        o_ref[...] = jnp.sum(p_ref[...], axis=1, keepdims=True)

    return pl.pallas_call(
        body,
        grid=(Vn // RT,),
        in_specs=[pl.BlockSpec((RT, Vn), lambda i: (i, 0))],
        out_specs=pl.BlockSpec((RT, 1), lambda i: (i, 0)),
        out_shape=jax.ShapeDtypeStruct((Vn, 1), jnp.float32),
    )(P)


def _combine(P16, xnb, xr, deg, xprev, mats, relu1, emit_t):
    Vn, BF = xr.shape
    F = mats[0][0].shape[0] if mats else BF
    B = BF // F
    RT = 256

    def body(*refs):
        it = iter(refs)
        p_ref = next(it)
        xn_ref = next(it)
        xr_ref = next(it)
        deg_ref = next(it)
        xp_ref = next(it) if xprev is not None else None
        wb_refs = []
        for _, bias, _dt in mats:
            wr = next(it)
            br = next(it) if bias is not None else None
            wb_refs.append((wr, br))
        outs = list(it)
        agg = jnp.dot(
            p_ref[...], xn_ref[...], preferred_element_type=jnp.float32
        )
        r = 1.0 / jnp.maximum(deg_ref[...], 1.0)
        t = xr_ref[...] + agg * r
        if relu1:
            t = jnp.maximum(t, 0.0)
        if xp_ref is not None:
            t = jnp.maximum(xp_ref[...] + t, 0.0)
        k = 0
        if emit_t:
            outs[0][...] = t
            k = 1
        for (wr, br), (_, _, dt), o_ref in zip(wb_refs, mats, outs[k:]):
            ys = []
            for b in range(B):
                y = jnp.dot(
                    t[:, b * F:(b + 1) * F], wr[...],
                    preferred_element_type=jnp.float32,
                )
                if br is not None:
                    y = y + br[...]
                ys.append(y.astype(dt))
            o_ref[...] = jnp.concatenate(ys, axis=1)

    in_specs = [
        pl.BlockSpec((RT, Vn), lambda i: (i, 0)),
        pl.BlockSpec((Vn, BF), lambda i: (0, 0)),
        pl.BlockSpec((RT, BF), lambda i: (i, 0)),
        pl.BlockSpec((RT, 1), lambda i: (i, 0)),
    ]
    args = [P16, xnb, xr, deg]
    if xprev is not None:
        in_specs.append(pl.BlockSpec((RT, xprev.shape[1]), lambda i: (i, 0)))
        args.append(xprev)
    out_shapes = []
    out_specs = []
    if emit_t:
        out_shapes.append(jax.ShapeDtypeStruct((Vn, BF), jnp.float32))
        out_specs.append(pl.BlockSpec((RT, BF), lambda i: (i, 0)))
    for W, bias, dt in mats:
        Ki, Fo = W.shape
        in_specs.append(pl.BlockSpec((Ki, Fo), lambda i: (0, 0)))
        args.append(W)
        if bias is not None:
            in_specs.append(pl.BlockSpec((1, Fo), lambda i: (0, 0)))
            args.append(bias.reshape(1, Fo))
        out_shapes.append(jax.ShapeDtypeStruct((Vn, B * Fo), dt))
        out_specs.append(pl.BlockSpec((RT, B * Fo), lambda i: (i, 0)))

    res = pl.pallas_call(
        body,
        grid=(Vn // RT,),
        in_specs=in_specs,
        out_specs=out_specs,
        out_shape=out_shapes,
    )(*args)
    return res if isinstance(res, (list, tuple)) else [res]


_EPAD = 128
_PASS_ROWS = 32


def _sc_build_p(dst, src, Vn):
    info = plsc.get_sparse_core_info()
    NC, NS = info.num_cores, info.num_subcores
    NW = NC * NS
    Ep = dst.shape[0]
    RPW = Vn // NW
    npass = RPW // _PASS_ROWS
    G = 8
    ng = Ep // (16 * G)
    mesh = plsc.VectorSubcoreMesh(core_axis_name="c", subcore_axis_name="s")

    @functools.partial(
        pl.kernel,
        out_type=jax.ShapeDtypeStruct((Vn * Vn,), jnp.float32),
        mesh=mesh,
        compiler_params=pltpu.CompilerParams(needs_layout_passes=False),
        scratch_types=[
            pltpu.VMEM((Ep,), jnp.int32),
            pltpu.VMEM((Ep,), jnp.int32),
            pltpu.VMEM((_PASS_ROWS * Vn,), jnp.float32),
            pltpu.SemaphoreType.DMA,
            pltpu.SemaphoreType.DMA,
        ],
    )
    def k(dst_hbm, src_hbm, out_hbm, dv, sv, acc, sem1, sem2):
        c = lax.axis_index("c")
        s = lax.axis_index("s")
        w = s * NC + c
        ones = jnp.ones((16,), jnp.float32)
        zero = jnp.zeros((16,), jnp.float32)

        cp1 = pltpu.async_copy(dst_hbm, dv, sem1)
        cp2 = pltpu.async_copy(src_hbm, sv, sem2)
        cp1.wait()
        cp2.wait()

        for p in range(npass):
            rowbase = (w * npass + p) * _PASS_ROWS

            def zf(i, _):
                for u in range(8):
                    acc[pl.ds(i * 128 + u * 16, 16)] = zero
                return 0

            lax.fori_loop(0, _PASS_ROWS * Vn // 128, zf, 0)

            def grp(i, _):
                for u in range(G):
                    off = i * (16 * G) + u * 16
                    dstv = dv[pl.ds(off, 16)]
                    srcv = sv[pl.ds(off, 16)]
                    loc = dstv - rowbase
                    m = (loc >= 0) & (loc < _PASS_ROWS)
                    idx = jnp.clip(loc * Vn + srcv, 0, _PASS_ROWS * Vn - 1)
                    plsc.addupdate_scatter(acc, [idx], ones, mask=m)
                return 0

            lax.fori_loop(0, ng, grp, 0)
            pltpu.sync_copy(
                acc, out_hbm.at[pl.ds(rowbase * Vn, _PASS_ROWS * Vn)]
            )

    return k(dst, src)


def kernel(vertices, edges, global_features, local_features, params):
    B, Vn, _ = vertices.shape
    E = edges.shape[1]
    BV = B * Vn
    p = params

    src = edges[0].astype(jnp.int32)
    dst = edges[1].astype(jnp.int32)
    pad = (-E) % _EPAD
    if pad:
        dst = jnp.pad(dst, (0, pad), constant_values=Vn)
        src = jnp.pad(src, (0, pad))
    P = _sc_build_p(dst, src, Vn).reshape(Vn, Vn)
    P16 = P.astype(jnp.bfloat16)
    deg = _rowsum(P)

    g = _split_fc(global_features, p["W_split"], p["b_split"])
    g3 = g.reshape(B, Vn, 128).transpose(1, 0, 2).reshape(BV, 128)
    v3 = vertices.transpose(1, 0, 2).reshape(BV, 3)
    xvg = jnp.pad(jnp.concatenate([v3, g3], axis=1), ((0, 0), (0, 125)))
    loc3 = local_features.transpose(1, 0, 2).reshape(BV, local_features.shape[2])
    c1 = p["conv1"]
    Wr_vg = jnp.pad(c1["Wr"][:131], ((0, 125), (0, 0)))
    Wn_vg = jnp.pad(c1["Wn"][:131], ((0, 125), (0, 0)))
    Wr_l = c1["Wr"][131:]
    Wn_l = c1["Wn"][131:]
    xr, xn = _conv1_mm(xvg, loc3, Wr_vg, Wn_vg, Wr_l, Wn_l, c1["b"])
    xr = xr.reshape(Vn, B * 256)
    xn = xn.reshape(Vn, B * 256)

    f32, bf16 = jnp.float32, jnp.bfloat16
    r0, r1 = p["res"]
    x2, xr_a, xn_a = _combine(
        P16, xn, xr, deg, None,
        [(r0["Wr1"], r0["b1"], f32), (r0["Wn1"], None, bf16)],
        relu1=True, emit_t=True)
    hr, hn = _combine(
        P16, xn_a, xr_a, deg, None,
        [(r0["Wr2"], r0["b2"], f32), (r0["Wn2"], None, bf16)],
        relu1=True, emit_t=False)
    x3, xr_b, xn_b = _combine(
        P16, hn, hr, deg, x2,
        [(r1["Wr1"], r1["b1"], f32), (r1["Wn1"], None, bf16)],
        relu1=False, emit_t=True)
    hr2, hn2 = _combine(
        P16, xn_b, xr_b, deg, None,
        [(r1["Wr2"], r1["b2"], f32), (r1["Wn2"], None, bf16)],
        relu1=True, emit_t=False)
    co = p["conv_out"]
    xr_o, xn_o = _combine(
        P16, hn2, hr2, deg, x3,
        [(co["Wr"], co["b"], f32), (co["Wn"], None, bf16)],
        relu1=False, emit_t=False)
    cf = p["conv_final"]
    Wr_f = jnp.pad(cf["Wr"], ((0, 0), (0, 13)))
    Wn_f = jnp.pad(cf["Wn"], ((0, 0), (0, 13)))
    b_f = jnp.pad(cf["b"], (0, 13))
    xr_f, xn_f = _combine(
        P16, xn_o, xr_o, deg, None,
        [(Wr_f, b_f, f32), (Wn_f, None, bf16)], relu1=True, emit_t=False)
    (out16,) = _combine(
        P16, xn_f, xr_f, deg, None, [], relu1=False, emit_t=True)
    return out16.reshape(Vn, B, 16)[:, :, :3].transpose(1, 0, 2)


# speedup vs baseline: 48.8857x; 1.0581x over previous
import functools

import jax
import jax.numpy as jnp
from jax import lax
from jax.experimental import pallas as pl
from jax.experimental.pallas import tpu as pltpu
from jax.experimental.pallas import tpu_sc as plsc


def _split_fc(gf, W, b):
    Bn, K = gf.shape
    N = W.shape[1]
    CB = 4096

    def body(g_ref, w_ref, b_ref, o_ref):
        o_ref[...] = (
            jnp.dot(g_ref[...], w_ref[...], preferred_element_type=jnp.float32)
            + b_ref[...]
        )

    return pl.pallas_call(
        body,
        grid=(N // CB,),
        in_specs=[
            pl.BlockSpec((Bn, K), lambda i: (0, 0)),
            pl.BlockSpec((K, CB), lambda i: (0, i)),
            pl.BlockSpec((1, CB), lambda i: (0, i)),
        ],
        out_specs=pl.BlockSpec((Bn, CB), lambda i: (0, i)),
        out_shape=jax.ShapeDtypeStruct((Bn, N), jnp.float32),
    )(gf, W, b.reshape(1, N))


def _conv1_mm(xvg, loc, Wr_vg, Wn_vg, Wr_l, Wn_l, b):
    BV, Kvg = xvg.shape
    Kl = loc.shape[1]
    F = Wr_l.shape[1]
    Vt = 512

    def body(x_ref, l_ref, wrvg, wnvg, wrl, wnl, b_ref, xr_ref, xn_ref):
        x = x_ref[...]
        l = l_ref[...]
        d = functools.partial(jnp.dot, preferred_element_type=jnp.float32)
        xr_ref[...] = (
            d(x, wrvg[...]) + d(l, wrl[...]) + b_ref[...]
        ).astype(jnp.bfloat16)
        xn_ref[...] = (d(x, wnvg[...]) + d(l, wnl[...])).astype(jnp.bfloat16)

    return pl.pallas_call(
        body,
        grid=(BV // Vt,),
        in_specs=[
            pl.BlockSpec((Vt, Kvg), lambda i: (i, 0)),
            pl.BlockSpec((Vt, Kl), lambda i: (i, 0)),
            pl.BlockSpec((Kvg, F), lambda i: (0, 0)),
            pl.BlockSpec((Kvg, F), lambda i: (0, 0)),
            pl.BlockSpec((Kl, F), lambda i: (0, 0)),
            pl.BlockSpec((Kl, F), lambda i: (0, 0)),
            pl.BlockSpec((1, F), lambda i: (0, 0)),
        ],
        out_specs=[
            pl.BlockSpec((Vt, F), lambda i: (i, 0)),
            pl.BlockSpec((Vt, F), lambda i: (i, 0)),
        ],
        out_shape=[
            jax.ShapeDtypeStruct((BV, F), jnp.bfloat16),
            jax.ShapeDtypeStruct((BV, F), jnp.bfloat16),
        ],
    )(xvg, loc, Wr_vg, Wn_vg, Wr_l, Wn_l, b.reshape(1, F))


def _combine(P16, xnb, xr, xprev, mats, relu1, emit_t, emit_dt=jnp.bfloat16):
    Vn, BF = xr.shape
    F = mats[0][0].shape[0] if mats else BF
    B = BF // F
    RT = 256

    def body(*refs):
        it = iter(refs)
        p_ref = next(it)
        xn_ref = next(it)
        xr_ref = next(it)
        xp_ref = next(it) if xprev is not None else None
        wb_refs = []
        for _, bias, _dt in mats:
            wr = next(it)
            br = next(it) if bias is not None else None
            wb_refs.append((wr, br))
        outs = list(it)
        pb = p_ref[...]
        agg = jnp.dot(pb, xn_ref[...], preferred_element_type=jnp.float32)
        deg = jnp.sum(pb.astype(jnp.float32), axis=1, keepdims=True)
        r = 1.0 / jnp.maximum(deg, 1.0)
        t = xr_ref[...].astype(jnp.float32) + agg * r
        if relu1:
            t = jnp.maximum(t, 0.0)
        if xp_ref is not None:
            t = jnp.maximum(xp_ref[...].astype(jnp.float32) + t, 0.0)
        k = 0
        if emit_t:
            outs[0][...] = t.astype(emit_dt)
            k = 1
        for (wr, br), (_, _, dt), o_ref in zip(wb_refs, mats, outs[k:]):
            ys = []
            for b in range(B):
                y = jnp.dot(
                    t[:, b * F:(b + 1) * F], wr[...],
                    preferred_element_type=jnp.float32,
                )
                if br is not None:
                    y = y + br[...]
                ys.append(y.astype(dt))
            o_ref[...] = jnp.concatenate(ys, axis=1)

    in_specs = [
        pl.BlockSpec((RT, Vn), lambda i: (i, 0)),
        pl.BlockSpec((Vn, BF), lambda i: (0, 0)),
        pl.BlockSpec((RT, BF), lambda i: (i, 0)),
    ]
    args = [P16, xnb, xr]
    if xprev is not None:
        in_specs.append(pl.BlockSpec((RT, xprev.shape[1]), lambda i: (i, 0)))
        args.append(xprev)
    out_shapes = []
    out_specs = []
    if emit_t:
        out_shapes.append(jax.ShapeDtypeStruct((Vn, BF), emit_dt))
        out_specs.append(pl.BlockSpec((RT, BF), lambda i: (i, 0)))
    for W, bias, dt in mats:
        Ki, Fo = W.shape
        in_specs.append(pl.BlockSpec((Ki, Fo), lambda i: (0, 0)))
        args.append(W)
        if bias is not None:
            in_specs.append(pl.BlockSpec((1, Fo), lambda i: (0, 0)))
            args.append(bias.reshape(1, Fo))
        out_shapes.append(jax.ShapeDtypeStruct((Vn, B * Fo), dt))
        out_specs.append(pl.BlockSpec((RT, B * Fo), lambda i: (i, 0)))

    res = pl.pallas_call(
        body,
        grid=(Vn // RT,),
        in_specs=in_specs,
        out_specs=out_specs,
        out_shape=out_shapes,
    )(*args)
    return res if isinstance(res, (list, tuple)) else [res]


_EPAD = 128
_PASS_ROWS = 32


def _sc_build_p(dst, src, Vn):
    info = plsc.get_sparse_core_info()
    NC, NS = info.num_cores, info.num_subcores
    NW = NC * NS
    Ep = dst.shape[0]
    RPW = Vn // NW
    npass = RPW // _PASS_ROWS
    G = 8
    ng = Ep // (16 * G)
    mesh = plsc.VectorSubcoreMesh(core_axis_name="c", subcore_axis_name="s")

    @functools.partial(
        pl.kernel,
        out_type=jax.ShapeDtypeStruct((Vn * Vn,), jnp.float32),
        mesh=mesh,
        compiler_params=pltpu.CompilerParams(needs_layout_passes=False),
        scratch_types=[
            pltpu.VMEM((Ep,), jnp.int32),
            pltpu.VMEM((Ep,), jnp.int32),
            pltpu.VMEM((_PASS_ROWS * Vn,), jnp.float32),
            pltpu.SemaphoreType.DMA,
            pltpu.SemaphoreType.DMA,
        ],
    )
    def k(dst_hbm, src_hbm, out_hbm, dv, sv, acc, sem1, sem2):
        c = lax.axis_index("c")
        s = lax.axis_index("s")
        w = s * NC + c
        ones = jnp.ones((16,), jnp.float32)
        zero = jnp.zeros((16,), jnp.float32)

        cp1 = pltpu.async_copy(dst_hbm, dv, sem1)
        cp2 = pltpu.async_copy(src_hbm, sv, sem2)
        cp1.wait()
        cp2.wait()

        for p in range(npass):
            rowbase = (w * npass + p) * _PASS_ROWS

            def zf(i, _):
                for u in range(8):
                    acc[pl.ds(i * 128 + u * 16, 16)] = zero
                return 0

            lax.fori_loop(0, _PASS_ROWS * Vn // 128, zf, 0)

            def grp(i, _):
                for u in range(G):
                    off = i * (16 * G) + u * 16
                    dstv = dv[pl.ds(off, 16)]
                    srcv = sv[pl.ds(off, 16)]
                    loc = dstv - rowbase
                    m = (loc >= 0) & (loc < _PASS_ROWS)
                    idx = jnp.clip(loc * Vn + srcv, 0, _PASS_ROWS * Vn - 1)
                    plsc.addupdate_scatter(acc, [idx], ones, mask=m)
                return 0

            lax.fori_loop(0, ng, grp, 0)
            pltpu.sync_copy(
                acc, out_hbm.at[pl.ds(rowbase * Vn, _PASS_ROWS * Vn)]
            )

    return k(dst, src)


def kernel(vertices, edges, global_features, local_features, params):
    B, Vn, _ = vertices.shape
    E = edges.shape[1]
    BV = B * Vn
    p = params

    src = edges[0].astype(jnp.int32)
    dst = edges[1].astype(jnp.int32)
    pad = (-E) % _EPAD
    if pad:
        dst = jnp.pad(dst, (0, pad), constant_values=Vn)
        src = jnp.pad(src, (0, pad))
    P = _sc_build_p(dst, src, Vn).reshape(Vn, Vn)
    P16 = P.astype(jnp.bfloat16)

    g = _split_fc(global_features, p["W_split"], p["b_split"])
    g3 = g.reshape(B, Vn, 128).transpose(1, 0, 2).reshape(BV, 128)
    v3 = vertices.transpose(1, 0, 2).reshape(BV, 3)
    xvg = jnp.pad(jnp.concatenate([v3, g3], axis=1), ((0, 0), (0, 125)))
    loc3 = local_features.transpose(1, 0, 2).reshape(BV, local_features.shape[2])
    c1 = p["conv1"]
    Wr_vg = jnp.pad(c1["Wr"][:131], ((0, 125), (0, 0)))
    Wn_vg = jnp.pad(c1["Wn"][:131], ((0, 125), (0, 0)))
    Wr_l = c1["Wr"][131:]
    Wn_l = c1["Wn"][131:]
    xr, xn = _conv1_mm(xvg, loc3, Wr_vg, Wn_vg, Wr_l, Wn_l, c1["b"])
    xr = xr.reshape(Vn, B * 256)
    xn = xn.reshape(Vn, B * 256)

    bf16 = jnp.bfloat16
    r0, r1 = p["res"]
    x2, xr_a, xn_a = _combine(
        P16, xn, xr, None,
        [(r0["Wr1"], r0["b1"], bf16), (r0["Wn1"], None, bf16)],
        relu1=True, emit_t=True)
    hr, hn = _combine(
        P16, xn_a, xr_a, None,
        [(r0["Wr2"], r0["b2"], bf16), (r0["Wn2"], None, bf16)],
        relu1=True, emit_t=False)
    x3, xr_b, xn_b = _combine(
        P16, hn, hr, x2,
        [(r1["Wr1"], r1["b1"], bf16), (r1["Wn1"], None, bf16)],
        relu1=False, emit_t=True)
    hr2, hn2 = _combine(
        P16, xn_b, xr_b, None,
        [(r1["Wr2"], r1["b2"], bf16), (r1["Wn2"], None, bf16)],
        relu1=True, emit_t=False)
    co = p["conv_out"]
    xr_o, xn_o = _combine(
        P16, hn2, hr2, x3,
        [(co["Wr"], co["b"], bf16), (co["Wn"], None, bf16)],
        relu1=False, emit_t=False)
    cf = p["conv_final"]
    Wr_f = jnp.pad(cf["Wr"], ((0, 0), (0, 13)))
    Wn_f = jnp.pad(cf["Wn"], ((0, 0), (0, 13)))
    b_f = jnp.pad(cf["b"], (0, 13))
    xr_f, xn_f = _combine(
        P16, xn_o, xr_o, None,
        [(Wr_f, b_f, bf16), (Wn_f, None, bf16)], relu1=True, emit_t=False)
    (out16,) = _combine(
        P16, xn_f, xr_f, None, [], relu1=False, emit_t=True,
        emit_dt=jnp.float32)
    return out16.reshape(Vn, B, 16)[:, :, :3].transpose(1, 0, 2)


# speedup vs baseline: 49.4966x; 1.0125x over previous
import functools

import jax
import jax.numpy as jnp
from jax import lax
from jax.experimental import pallas as pl
from jax.experimental.pallas import tpu as pltpu
from jax.experimental.pallas import tpu_sc as plsc


def _split_fc(gf, W, b):
    Bn, K = gf.shape
    N = W.shape[1]
    CB = 8192

    def body(g_ref, w_ref, b_ref, o_ref):
        o_ref[...] = (
            jnp.dot(g_ref[...], w_ref[...], preferred_element_type=jnp.float32)
            + b_ref[...]
        )

    return pl.pallas_call(
        body,
        grid=(N // CB,),
        in_specs=[
            pl.BlockSpec((Bn, K), lambda i: (0, 0)),
            pl.BlockSpec((K, CB), lambda i: (0, i)),
            pl.BlockSpec((1, CB), lambda i: (0, i)),
        ],
        out_specs=pl.BlockSpec((Bn, CB), lambda i: (0, i)),
        out_shape=jax.ShapeDtypeStruct((Bn, N), jnp.float32),
    )(gf, W, b.reshape(1, N))


def _conv1_mm(xvg, loc, Wr_vg, Wn_vg, Wr_l, Wn_l, b):
    BV, Kvg = xvg.shape
    Kl = loc.shape[1]
    F = Wr_l.shape[1]
    Vt = 512

    def body(x_ref, l_ref, wrvg, wnvg, wrl, wnl, b_ref, xr_ref, xn_ref):
        x = x_ref[...]
        l = l_ref[...]
        d = functools.partial(jnp.dot, preferred_element_type=jnp.float32)
        xr_ref[...] = (
            d(x, wrvg[...]) + d(l, wrl[...]) + b_ref[...]
        ).astype(jnp.bfloat16)
        xn_ref[...] = (d(x, wnvg[...]) + d(l, wnl[...])).astype(jnp.bfloat16)

    return pl.pallas_call(
        body,
        grid=(BV // Vt,),
        in_specs=[
            pl.BlockSpec((Vt, Kvg), lambda i: (i, 0)),
            pl.BlockSpec((Vt, Kl), lambda i: (i, 0)),
            pl.BlockSpec((Kvg, F), lambda i: (0, 0)),
            pl.BlockSpec((Kvg, F), lambda i: (0, 0)),
            pl.BlockSpec((Kl, F), lambda i: (0, 0)),
            pl.BlockSpec((Kl, F), lambda i: (0, 0)),
            pl.BlockSpec((1, F), lambda i: (0, 0)),
        ],
        out_specs=[
            pl.BlockSpec((Vt, F), lambda i: (i, 0)),
            pl.BlockSpec((Vt, F), lambda i: (i, 0)),
        ],
        out_shape=[
            jax.ShapeDtypeStruct((BV, F), jnp.bfloat16),
            jax.ShapeDtypeStruct((BV, F), jnp.bfloat16),
        ],
    )(xvg, loc, Wr_vg, Wn_vg, Wr_l, Wn_l, b.reshape(1, F))


def _combine(P16, xnb, xr, xprev, mats, relu1, emit_t, emit_dt=jnp.bfloat16):
    Vn, BF = xr.shape
    F = mats[0][0].shape[0] if mats else BF
    B = BF // F
    RT = 256

    def body(*refs):
        it = iter(refs)
        p_ref = next(it)
        xn_ref = next(it)
        xr_ref = next(it)
        xp_ref = next(it) if xprev is not None else None
        wb_refs = []
        for _, bias, _dt in mats:
            wr = next(it)
            br = next(it) if bias is not None else None
            wb_refs.append((wr, br))
        outs = list(it)
        pb = p_ref[...]
        agg = jnp.dot(pb, xn_ref[...], preferred_element_type=jnp.float32)
        deg = jnp.sum(pb.astype(jnp.float32), axis=1, keepdims=True)
        r = 1.0 / jnp.maximum(deg, 1.0)
        t = xr_ref[...].astype(jnp.float32) + agg * r
        if relu1:
            t = jnp.maximum(t, 0.0)
        if xp_ref is not None:
            t = jnp.maximum(xp_ref[...].astype(jnp.float32) + t, 0.0)
        k = 0
        if emit_t:
            outs[0][...] = t.astype(emit_dt)
            k = 1
        for (wr, br), (_, _, dt), o_ref in zip(wb_refs, mats, outs[k:]):
            ys = []
            for b in range(B):
                y = jnp.dot(
                    t[:, b * F:(b + 1) * F], wr[...],
                    preferred_element_type=jnp.float32,
                )
                if br is not None:
                    y = y + br[...]
                ys.append(y.astype(dt))
            o_ref[...] = jnp.concatenate(ys, axis=1)

    in_specs = [
        pl.BlockSpec((RT, Vn), lambda i: (i, 0)),
        pl.BlockSpec((Vn, BF), lambda i: (0, 0)),
        pl.BlockSpec((RT, BF), lambda i: (i, 0)),
    ]
    args = [P16, xnb, xr]
    if xprev is not None:
        in_specs.append(pl.BlockSpec((RT, xprev.shape[1]), lambda i: (i, 0)))
        args.append(xprev)
    out_shapes = []
    out_specs = []
    if emit_t:
        out_shapes.append(jax.ShapeDtypeStruct((Vn, BF), emit_dt))
        out_specs.append(pl.BlockSpec((RT, BF), lambda i: (i, 0)))
    for W, bias, dt in mats:
        Ki, Fo = W.shape
        in_specs.append(pl.BlockSpec((Ki, Fo), lambda i: (0, 0)))
        args.append(W)
        if bias is not None:
            in_specs.append(pl.BlockSpec((1, Fo), lambda i: (0, 0)))
            args.append(bias.reshape(1, Fo))
        out_shapes.append(jax.ShapeDtypeStruct((Vn, B * Fo), dt))
        out_specs.append(pl.BlockSpec((RT, B * Fo), lambda i: (i, 0)))

    res = pl.pallas_call(
        body,
        grid=(Vn // RT,),
        in_specs=in_specs,
        out_specs=out_specs,
        out_shape=out_shapes,
    )(*args)
    return res if isinstance(res, (list, tuple)) else [res]


_EPAD = 128
_PASS_ROWS = 32


def _sc_build_p(dst, src, Vn):
    info = plsc.get_sparse_core_info()
    NC, NS = info.num_cores, info.num_subcores
    NW = NC * NS
    Ep = dst.shape[0]
    RPW = Vn // NW
    npass = RPW // _PASS_ROWS
    G = 8
    ng = Ep // (16 * G)
    mesh = plsc.VectorSubcoreMesh(core_axis_name="c", subcore_axis_name="s")

    @functools.partial(
        pl.kernel,
        out_type=jax.ShapeDtypeStruct((Vn * Vn,), jnp.float32),
        mesh=mesh,
        compiler_params=pltpu.CompilerParams(needs_layout_passes=False),
        scratch_types=[
            pltpu.VMEM((Ep,), jnp.int32),
            pltpu.VMEM((Ep,), jnp.int32),
            pltpu.VMEM((_PASS_ROWS * Vn,), jnp.float32),
            pltpu.SemaphoreType.DMA,
            pltpu.SemaphoreType.DMA,
        ],
    )
    def k(dst_hbm, src_hbm, out_hbm, dv, sv, acc, sem1, sem2):
        c = lax.axis_index("c")
        s = lax.axis_index("s")
        w = s * NC + c
        ones = jnp.ones((16,), jnp.float32)
        zero = jnp.zeros((16,), jnp.float32)

        cp1 = pltpu.async_copy(dst_hbm, dv, sem1)
        cp2 = pltpu.async_copy(src_hbm, sv, sem2)
        cp1.wait()
        cp2.wait()

        for p in range(npass):
            rowbase = (w * npass + p) * _PASS_ROWS

            def zf(i, _):
                for u in range(8):
                    acc[pl.ds(i * 128 + u * 16, 16)] = zero
                return 0

            lax.fori_loop(0, _PASS_ROWS * Vn // 128, zf, 0)

            def grp(i, _):
                for u in range(G):
                    off = i * (16 * G) + u * 16
                    dstv = dv[pl.ds(off, 16)]
                    srcv = sv[pl.ds(off, 16)]
                    loc = dstv - rowbase
                    m = (loc >= 0) & (loc < _PASS_ROWS)
                    idx = jnp.clip(loc * Vn + srcv, 0, _PASS_ROWS * Vn - 1)
                    plsc.addupdate_scatter(acc, [idx], ones, mask=m)
                return 0

            lax.fori_loop(0, ng, grp, 0)
            pltpu.sync_copy(
                acc, out_hbm.at[pl.ds(rowbase * Vn, _PASS_ROWS * Vn)]
            )

    return k(dst, src)


def kernel(vertices, edges, global_features, local_features, params):
    B, Vn, _ = vertices.shape
    E = edges.shape[1]
    BV = B * Vn
    p = params

    src = edges[0].astype(jnp.int32)
    dst = edges[1].astype(jnp.int32)
    pad = (-E) % _EPAD
    if pad:
        dst = jnp.pad(dst, (0, pad), constant_values=Vn)
        src = jnp.pad(src, (0, pad))
    P = _sc_build_p(dst, src, Vn).reshape(Vn, Vn)
    P16 = P.astype(jnp.bfloat16)

    g = _split_fc(global_features, p["W_split"], p["b_split"])
    g3 = g.reshape(B, Vn, 128).transpose(1, 0, 2).reshape(BV, 128)
    v3 = vertices.transpose(1, 0, 2).reshape(BV, 3)
    xvg = jnp.pad(jnp.concatenate([v3, g3], axis=1), ((0, 0), (0, 125)))
    loc3 = local_features.transpose(1, 0, 2).reshape(BV, local_features.shape[2])
    c1 = p["conv1"]
    Wr_vg = jnp.pad(c1["Wr"][:131], ((0, 125), (0, 0)))
    Wn_vg = jnp.pad(c1["Wn"][:131], ((0, 125), (0, 0)))
    Wr_l = c1["Wr"][131:]
    Wn_l = c1["Wn"][131:]
    xr, xn = _conv1_mm(xvg, loc3, Wr_vg, Wn_vg, Wr_l, Wn_l, c1["b"])
    xr = xr.reshape(Vn, B * 256)
    xn = xn.reshape(Vn, B * 256)

    bf16 = jnp.bfloat16
    r0, r1 = p["res"]
    x2, xr_a, xn_a = _combine(
        P16, xn, xr, None,
        [(r0["Wr1"], r0["b1"], bf16), (r0["Wn1"], None, bf16)],
        relu1=True, emit_t=True)
    hr, hn = _combine(
        P16, xn_a, xr_a, None,
        [(r0["Wr2"], r0["b2"], bf16), (r0["Wn2"], None, bf16)],
        relu1=True, emit_t=False)
    x3, xr_b, xn_b = _combine(
        P16, hn, hr, x2,
        [(r1["Wr1"], r1["b1"], bf16), (r1["Wn1"], None, bf16)],
        relu1=False, emit_t=True)
    hr2, hn2 = _combine(
        P16, xn_b, xr_b, None,
        [(r1["Wr2"], r1["b2"], bf16), (r1["Wn2"], None, bf16)],
        relu1=True, emit_t=False)
    co = p["conv_out"]
    xr_o, xn_o = _combine(
        P16, hn2, hr2, x3,
        [(co["Wr"], co["b"], bf16), (co["Wn"], None, bf16)],
        relu1=False, emit_t=False)
    cf = p["conv_final"]
    Wr_f = jnp.pad(cf["Wr"], ((0, 0), (0, 13)))
    Wn_f = jnp.pad(cf["Wn"], ((0, 0), (0, 13)))
    b_f = jnp.pad(cf["b"], (0, 13))
    xr_f, xn_f = _combine(
        P16, xn_o, xr_o, None,
        [(Wr_f, b_f, bf16), (Wn_f, None, bf16)], relu1=True, emit_t=False)
    (out16,) = _combine(
        P16, xn_f, xr_f, None, [], relu1=False, emit_t=True,
        emit_dt=jnp.float32)
    return out16.reshape(Vn, B, 16)[:, :, :3].transpose(1, 0, 2)
